# baseline (speedup 1.0000x reference)
import functools

import jax
import jax.numpy as jnp
from jax import lax
from jax.experimental import pallas as pl
from jax.experimental.pallas import tpu as pltpu
from jax.experimental.pallas import tpu_sc as plsc

B = 16384
D = 64
V = 1000000
NC = 2
NS = 16
NW = NC * NS
BPW = B // NW
CH = 128
NCH = BPW // CH

TW = 16384
KSC = 10
VSC = KSC * TW
VTC = V - VSC
TGC = (VTC + TW - 1) // TW
NCOL = VSC // CH // NW

_sc_mesh = plsc.VectorSubcoreMesh(core_axis_name="c", subcore_axis_name="s")


def _pack_body(ta, tb, tc_, td, out):
    ident = (jax.lax.broadcasted_iota(jnp.int32, (D, D), 0)
             == jax.lax.broadcasted_iota(jnp.int32, (D, D), 1)
             ).astype(jnp.bfloat16)
    dn = (((0,), (0,)), ((), ()))
    cols = []
    for t in (ta, tb, tc_, td):
        x = jax.lax.dot_general(t[...].astype(jnp.bfloat16), ident, dn,
                                preferred_element_type=jnp.float32
                                ).astype(jnp.bfloat16)
        cols.append(jax.lax.bitcast_convert_type(x, jnp.uint16).astype(jnp.uint32))
    u = cols[0] | (cols[1] << 16)
    i = cols[2] | (cols[3] << 16)
    out[:, :D] = jax.lax.bitcast_convert_type(u, jnp.float32)
    out[:, D:] = jax.lax.bitcast_convert_type(i, jnp.float32)


def _tc_pack(ua_t, ub_t, ia_t, ib_t):
    blk = pl.BlockSpec((D, TW), lambda i: (0, i + KSC))
    return pl.pallas_call(
        _pack_body,
        grid=(TGC,),
        in_specs=[blk, blk, blk, blk],
        out_specs=pl.BlockSpec((TW, 2 * D), lambda i: (i, 0)),
        out_shape=jax.ShapeDtypeStruct((VTC, 2 * D), jnp.float32),
    )(ua_t, ub_t, ia_t, ib_t)


@functools.partial(
    pl.kernel,
    mesh=_sc_mesh,
    compiler_params=pltpu.CompilerParams(needs_layout_passes=False),
    out_type=[jax.ShapeDtypeStruct((VSC, 2 * D), jnp.float32) for _ in range(2)],
    scratch_types=[
        pltpu.VMEM((D, CH), jnp.float32),
        pltpu.VMEM((D, CH), jnp.float32),
        pltpu.VMEM((D, CH), jnp.float32),
        pltpu.VMEM((D, CH), jnp.float32),
        pltpu.VMEM((CH, 2 * D), jnp.float32),
        pltpu.VMEM((CH, 2 * D), jnp.float32),
        pltpu.SemaphoreType.DMA,
        pltpu.SemaphoreType.DMA,
        pltpu.SemaphoreType.DMA,
        pltpu.SemaphoreType.DMA,
    ],
)
def _sc_pack(tum, tug, tim, tig, o_user, o_item,
             sA0, sA1, sB0, sB1, obA, obB, fsA, fsB, wsA, wsB):
    wid = lax.axis_index("s") * NC + lax.axis_index("c")
    col0 = wid * NCOL
    iotav = lax.broadcasted_iota(jnp.int32, (16,), 0)
    rvecs = [(rg * 16 + iotav) for rg in range(8)]
    kzero = jnp.zeros((16,), jnp.int32)

    def fetch(c, t0, t1, d0, d1, sem):
        src = pl.ds(pl.multiple_of((col0 + c) * CH, CH), CH)
        pltpu.async_copy(t0.at[:, src], d0, sem)
        pltpu.async_copy(t1.at[:, src], d1, sem)

    def drain_fetch(t0, t1, d0, d1, sem):
        pltpu.make_async_copy(t0.at[:, pl.ds(0, CH)], d0, sem).wait()
        pltpu.make_async_copy(t1.at[:, pl.ds(0, CH)], d1, sem).wait()

    def transpose_pair(s0, s1, ob):
        for k in range(D):
            for rg in range(8):
                rv = rvecs[rg]
                plsc.store_scatter(ob, [rv, kzero + k], s0[k, pl.ds(rg * 16, 16)])
                plsc.store_scatter(ob, [rv, kzero + (D + k)], s1[k, pl.ds(rg * 16, 16)])

    fetch(0, tum, tug, sA0, sA1, fsA)
    fetch(0, tim, tig, sB0, sB1, fsB)

    def body(c, carry):
        dst = pl.ds(pl.multiple_of((col0 + c) * CH, CH), CH)

        @pl.when(c > 0)
        def _():
            pltpu.make_async_copy(obA, o_user.at[pl.ds(0, CH)], wsA).wait()
        drain_fetch(tum, tug, sA0, sA1, fsA)
        transpose_pair(sA0, sA1, obA)

        @pl.when(c + 1 < NCOL)
        def _():
            fetch(c + 1, tum, tug, sA0, sA1, fsA)
        pltpu.async_copy(obA, o_user.at[dst], wsA)

        @pl.when(c > 0)
        def _():
            pltpu.make_async_copy(obB, o_item.at[pl.ds(0, CH)], wsB).wait()
        drain_fetch(tim, tig, sB0, sB1, fsB)
        transpose_pair(sB0, sB1, obB)

        @pl.when(c + 1 < NCOL)
        def _():
            fetch(c + 1, tim, tig, sB0, sB1, fsB)
        pltpu.async_copy(obB, o_item.at[dst], wsB)
        return carry

    lax.fori_loop(0, NCOL, body, 0)
    pltpu.make_async_copy(obA, o_user.at[pl.ds(0, CH)], wsA).wait()
    pltpu.make_async_copy(obB, o_item.at[pl.ds(0, CH)], wsB).wait()


@functools.partial(
    pl.kernel,
    mesh=_sc_mesh,
    out_type=[jax.ShapeDtypeStruct((B, 2 * D), jnp.float32) for _ in range(4)],
    scratch_types=[
        pltpu.VMEM((NCH, CH), jnp.int32),
        pltpu.VMEM((NCH, CH), jnp.int32),
        pltpu.VMEM((NCH, CH), jnp.int32),
        pltpu.VMEM((CH, 2 * D), jnp.float32),
        pltpu.VMEM((CH, 2 * D), jnp.float32),
        pltpu.VMEM((CH, 2 * D), jnp.float32),
        pltpu.VMEM((CH, 2 * D), jnp.float32),
        pltpu.SemaphoreType.DMA,
        pltpu.SemaphoreType.DMA,
    ],
)
def _sc_gather(users3d, items3d, mega_tc, user_sc, item_sc,
               o_uA, o_uB, o_iA, o_iB,
               idx, idxA, idxB, bA0, bA1, bB0, bB1, gsem, wsem):
    wid = lax.axis_index("s") * NC + lax.axis_index("c")
    base = wid * BPW
    bufsA = (bA0, bA1)
    bufsB = (bB0, bB1)
    wpend = []
    for idx3d, sc_tab, outA, outB in ((users3d, user_sc, o_uA, o_uB),
                                      (items3d, item_sc, o_iA, o_iB)):
        pltpu.sync_copy(idx3d.at[wid], idx)
        for c4 in range(NCH):
            for g in range(8):
                sl = pl.ds(g * 16, 16)
                v = idx[c4, sl]
                idxA[c4, sl] = jnp.clip(v - VSC, 0, VTC - 1)
                idxB[c4, sl] = jnp.minimum(v, VSC - 1)
        for c in range(NCH):
            par = c % 2
            bA, bB = bufsA[par], bufsB[par]
            if len(wpend) >= 2:
                for h in wpend.pop(0):
                    h.wait()
            hA = pltpu.async_copy(mega_tc.at[idxA.at[c]], bA, gsem)
            hB = pltpu.async_copy(sc_tab.at[idxB.at[c]], bB, gsem)
            hA.wait()
            hB.wait()
            dst = pl.ds(base + c * CH, CH)
            wpend.append([pltpu.async_copy(bA, outA.at[dst], wsem),
                          pltpu.async_copy(bB, outB.at[dst], wsem)])
    for hs in wpend:
        for h in hs:
            h.wait()


BB = 2048


def _unpack(packed):
    w = jax.lax.bitcast_convert_type(packed, jnp.uint32)
    lo = jax.lax.bitcast_convert_type((w & 0xFFFF).astype(jnp.uint16), jnp.bfloat16)
    hi = jax.lax.bitcast_convert_type((w >> 16).astype(jnp.uint16), jnp.bfloat16)
    return lo, hi


def _mlp_body(euA, euB, eiA, eiB, us, it, w1a, w1b, b1, w2, b2, w3, b3,
              wg, bg, wfa, wfb, bfv, out):
    f32 = jnp.float32
    mu = us[...] < VSC
    mi = it[...] < VSC
    ulo, uhi = _unpack(euA[:, :D])
    ilo, ihi = _unpack(eiA[:, D:])
    um = jnp.where(mu, euB[:, :D], ulo.astype(f32))
    ug = jnp.where(mu, euB[:, D:], uhi.astype(f32))
    im = jnp.where(mi, eiB[:, :D], ilo.astype(f32))
    ig = jnp.where(mi, eiB[:, D:], ihi.astype(f32))
    h = jnp.dot(um, w1a[...], preferred_element_type=f32)
    h = h + jnp.dot(im, w1b[...], preferred_element_type=f32) + b1[...]
    h = jnp.maximum(h, 0.0)
    h = jnp.maximum(jnp.dot(h, w2[...], preferred_element_type=f32) + b2[...], 0.0)
    h = jnp.dot(h, w3[...], preferred_element_type=f32) + b3[...]
    g = ug * ig
    og = jnp.dot(g, wg[...], preferred_element_type=f32) + bg[...]
    o = jnp.sum(h * wfa[...], axis=1) + jnp.sum(og * wfb[...], axis=1)
    out[...] = o + bfv[0, 0]


def _mlp(euA, euB, eiA, eiB, us, it,
         w1a, w1b, b1, w2, b2, w3, b3, wg, bg, wfa, wfb, bfv):
    full = lambda shape: pl.BlockSpec(shape, lambda i: (0,) * len(shape))
    blk = pl.BlockSpec((BB, 2 * D), lambda i: (i, 0))
    iblk = pl.BlockSpec((BB, 1), lambda i: (i, 0))
    return pl.pallas_call(
        _mlp_body,
        grid=(B // BB,),
        in_specs=[
            blk, blk, blk, blk, iblk, iblk,
            full((D, D)), full((D, D)), full((1, D)),
            full((D, D)), full((1, D)),
            full((D, D // 2)), full((1, D // 2)),
            full((D, D // 2)), full((1, D // 2)),
            full((1, D // 2)), full((1, D // 2)), full((1, 1)),
        ],
        out_specs=pl.BlockSpec((BB,), lambda i: (i,)),
        out_shape=jax.ShapeDtypeStruct((B,), jnp.float32),
    )(euA, euB, eiA, eiB, us, it,
      w1a, w1b, b1, w2, b2, w3, b3, wg, bg, wfa, wfb, bfv)


def kernel(users, items, ue_mlp, ie_mlp, ue_gmf, ie_gmf,
           W_gmf, b_gmf, W1, b1, W2, b2, W3, b3, Wf, bf):
    users = users.astype(jnp.int32)
    items = items.astype(jnp.int32)
    users3d = users.reshape(NW, NCH, CH)
    items3d = items.reshape(NW, NCH, CH)
    mega_tc = _tc_pack(ue_mlp.T, ue_gmf.T, ie_mlp.T, ie_gmf.T)
    user_sc, item_sc = _sc_pack(ue_mlp.T, ue_gmf.T, ie_mlp.T, ie_gmf.T)
    ouA, ouB, oiA, oiB = _sc_gather(users3d, items3d, mega_tc, user_sc, item_sc)
    return _mlp(
        ouA, ouB, oiA, oiB, users.reshape(B, 1), items.reshape(B, 1),
        W1[:, :D].T, W1[:, D:].T, b1.reshape(1, D),
        W2.T, b2.reshape(1, D),
        W3.T, b3.reshape(1, D // 2),
        W_gmf.T, b_gmf.reshape(1, D // 2),
        Wf[:, :D // 2], Wf[:, D // 2:], bf.reshape(1, 1),
    )


# speedup vs baseline: 3.4899x; 3.4899x over previous
import functools

import jax
import jax.numpy as jnp
from jax import lax
from jax.experimental import pallas as pl
from jax.experimental.pallas import tpu as pltpu
from jax.experimental.pallas import tpu_sc as plsc

B = 16384
D = 64
V = 1000000
NC = 2
NS = 16
NW = NC * NS
BPW = B // NW
CH = 128
NCH = BPW // CH

TW = 16384
TG = (V + TW - 1) // TW

_sc_mesh = plsc.VectorSubcoreMesh(core_axis_name="c", subcore_axis_name="s")


def _pack_body(ta, tb, tc_, td, out):
    ident = (jax.lax.broadcasted_iota(jnp.int32, (D, D), 0)
             == jax.lax.broadcasted_iota(jnp.int32, (D, D), 1)
             ).astype(jnp.bfloat16)
    dn = (((0,), (0,)), ((), ()))
    cols = []
    for t in (ta, tb, tc_, td):
        x = jax.lax.dot_general(t[...].astype(jnp.bfloat16), ident, dn,
                                preferred_element_type=jnp.float32
                                ).astype(jnp.bfloat16)
        cols.append(jax.lax.bitcast_convert_type(x, jnp.uint16).astype(jnp.uint32))
    u = cols[0] | (cols[1] << 16)
    i = cols[2] | (cols[3] << 16)
    out[:, :D] = jax.lax.bitcast_convert_type(u, jnp.float32)
    out[:, D:] = jax.lax.bitcast_convert_type(i, jnp.float32)


def _mega_pack(ua_t, ub_t, ia_t, ib_t):
    blk = pl.BlockSpec((D, TW), lambda i: (0, i))
    return pl.pallas_call(
        _pack_body,
        grid=(TG,),
        in_specs=[blk, blk, blk, blk],
        out_specs=pl.BlockSpec((TW, 2 * D), lambda i: (i, 0)),
        out_shape=jax.ShapeDtypeStruct((V, 2 * D), jnp.float32),
    )(ua_t, ub_t, ia_t, ib_t)


@functools.partial(
    pl.kernel,
    mesh=_sc_mesh,
    out_type=[jax.ShapeDtypeStruct((B, 2 * D), jnp.float32) for _ in range(2)],
    scratch_types=[
        pltpu.VMEM((NCH, CH), jnp.int32),
        pltpu.VMEM((BPW, 2 * D), jnp.float32),
        pltpu.SemaphoreType.DMA,
    ],
)
def _sc_gather(users3d, items3d, tab, o_u, o_i, idx, buf, sem):
    wid = lax.axis_index("s") * NC + lax.axis_index("c")
    base = wid * BPW
    for idx3d, out in ((users3d, o_u), (items3d, o_i)):
        pltpu.sync_copy(idx3d.at[wid], idx)
        handles = [
            pltpu.async_copy(tab.at[idx.at[c]], buf.at[pl.ds(c * CH, CH)], sem)
            for c in range(NCH)
        ]
        for h in handles:
            h.wait()
        pltpu.sync_copy(buf, out.at[pl.ds(base, BPW)])


BB = 2048


def _unpack(packed):
    w = jax.lax.bitcast_convert_type(packed, jnp.uint32)
    lo = jax.lax.bitcast_convert_type((w & 0xFFFF).astype(jnp.uint16), jnp.bfloat16)
    hi = jax.lax.bitcast_convert_type((w >> 16).astype(jnp.uint16), jnp.bfloat16)
    return lo, hi


def _mlp_body(eu, ei, w1a, w1b, b1, w2, b2, w3, b3, wg, bg, wfa, wfb, bfv, out):
    f32 = jnp.float32
    um, ug = _unpack(eu[:, :D])
    im, ig = _unpack(ei[:, D:])
    h = jnp.dot(um, w1a[...], preferred_element_type=f32)
    h = h + jnp.dot(im, w1b[...], preferred_element_type=f32) + b1[...]
    h = jnp.maximum(h, 0.0)
    h = jnp.maximum(jnp.dot(h, w2[...], preferred_element_type=f32) + b2[...], 0.0)
    h = jnp.dot(h, w3[...], preferred_element_type=f32) + b3[...]
    g = ug.astype(f32) * ig.astype(f32)
    og = jnp.dot(g, wg[...], preferred_element_type=f32) + bg[...]
    o = jnp.sum(h * wfa[...], axis=1) + jnp.sum(og * wfb[...], axis=1)
    out[...] = o + bfv[0, 0]


def _mlp(eu, ei, w1a, w1b, b1, w2, b2, w3, b3, wg, bg, wfa, wfb, bfv):
    full = lambda shape: pl.BlockSpec(shape, lambda i: (0,) * len(shape))
    blk = pl.BlockSpec((BB, 2 * D), lambda i: (i, 0))
    return pl.pallas_call(
        _mlp_body,
        grid=(B // BB,),
        in_specs=[
            blk, blk,
            full((D, D)), full((D, D)), full((1, D)),
            full((D, D)), full((1, D)),
            full((D, D // 2)), full((1, D // 2)),
            full((D, D // 2)), full((1, D // 2)),
            full((1, D // 2)), full((1, D // 2)), full((1, 1)),
        ],
        out_specs=pl.BlockSpec((BB,), lambda i: (i,)),
        out_shape=jax.ShapeDtypeStruct((B,), jnp.float32),
    )(eu, ei, w1a, w1b, b1, w2, b2, w3, b3, wg, bg, wfa, wfb, bfv)


def kernel(users, items, ue_mlp, ie_mlp, ue_gmf, ie_gmf,
           W_gmf, b_gmf, W1, b1, W2, b2, W3, b3, Wf, bf):
    users3d = users.astype(jnp.int32).reshape(NW, NCH, CH)
    items3d = items.astype(jnp.int32).reshape(NW, NCH, CH)
    tab = _mega_pack(ue_mlp.T, ue_gmf.T, ie_mlp.T, ie_gmf.T)
    eu, ei = _sc_gather(users3d, items3d, tab)
    return _mlp(
        eu, ei,
        W1[:, :D].T.astype(jnp.bfloat16), W1[:, D:].T.astype(jnp.bfloat16),
        b1.reshape(1, D),
        W2.T, b2.reshape(1, D),
        W3.T, b3.reshape(1, D // 2),
        W_gmf.T, b_gmf.reshape(1, D // 2),
        Wf[:, :D // 2], Wf[:, D // 2:], bf.reshape(1, 1),
    )
